# baseline (speedup 1.0000x reference)
import jax
import jax.numpy as jnp
from jax import lax
from jax.experimental import pallas as pl
from jax.experimental.pallas import tpu as pltpu
from jax.experimental.pallas import tpu_sc as plsc

_NC = 2
_NS = 16
_NW = _NC * _NS
_L = 16

_B = 16384
_D = 128
_BPW = _B // _NW
_CH = 256
_OFFS = (0, 32, 288, 320, 384)
_VTOT = 416


def _body(idx_hbm, tab_hbm, out_hbm, tab_v, idx_v, out_v, bs_v):
    wid = lax.axis_index("s") * _NC + lax.axis_index("c")
    base = wid * _BPW

    pltpu.sync_copy(tab_hbm, tab_v)
    for t in range(5):
        pltpu.sync_copy(
            idx_hbm.at[pl.ds(t * _B + base, _BPW)],
            idx_v.at[pl.ds(t * _BPW, _BPW)],
        )

    lane = lax.iota(jnp.int32, _L)
    cvecs = [lane + u * _L for u in range(_D // (2 * _L))]

    @plsc.parallel_loop(0, _BPW // _L, unroll=2)
    def _(g):
        slot = (g & 1) * 5 * _L
        for t in range(5):
            bv = (idx_v[pl.ds(t * _BPW + g * _L, _L)] + _OFFS[t]) * (_D // 2)
            bs_v[pl.ds(slot + t * _L, _L)] = bv
        for j in range(_L):
            ob = (g * _L + j) * _D
            bspl = [
                plsc.load_gather(
                    bs_v, [jnp.full((_L,), slot + t * _L + j, jnp.int32)]
                )
                for t in range(5)
            ]
            vs = [
                plsc.bitcast(
                    plsc.load_gather(tab_v, [bspl[t] + cvecs[u]]), jnp.bfloat16
                )
                for u in range(_D // (2 * _L))
                for t in range(5)
            ]
            for u in range(_D // (2 * _L)):
                c = vs[5 * u : 5 * u + 5]
                acc = ((c[0] + c[1]) + (c[2] + c[3])) + c[4]
                a, b = plsc.unpack(acc, format=plsc.PackFormat.INTERLEAVED)
                out_v[pl.ds(ob + u * 2 * _L, _L)] = a
                out_v[pl.ds(ob + u * 2 * _L + _L, _L)] = b
    pltpu.sync_copy(out_v, out_hbm.at[pl.ds(base * _D, _BPW * _D)])


@jax.jit
def kernel(x_features_indices, W_exchange, W_pair, W_type, W_feature, W_level):
    idx = x_features_indices.astype(jnp.int32).T.reshape(-1)
    tab = (
        jnp.concatenate([W_exchange, W_pair, W_type, W_feature, W_level], axis=0)
        .astype(jnp.bfloat16)
        .reshape(_VTOT, _D // (2 * _L), 2, _L)
        .transpose(0, 1, 3, 2)
        .reshape(-1, 2)
    )
    tab = lax.bitcast_convert_type(tab, jnp.int32)
    mesh = plsc.VectorSubcoreMesh(
        core_axis_name="c", subcore_axis_name="s",
        num_cores=_NC, num_subcores=_NS,
    )
    f = pl.kernel(
        _body,
        out_type=jax.ShapeDtypeStruct((_B * _D,), jnp.float32),
        mesh=mesh,
        compiler_params=pltpu.CompilerParams(needs_layout_passes=False),
        scratch_types=[
            pltpu.VMEM((_VTOT * _D // 2,), jnp.int32),
            pltpu.VMEM((5 * _BPW,), jnp.int32),
            pltpu.VMEM((_BPW * _D,), jnp.float32),
            pltpu.VMEM((2 * 5 * _L,), jnp.int32),
        ],
    )
    return f(idx, tab).reshape(_B, _D)


# speedup vs baseline: 1.1138x; 1.1138x over previous
import jax
import jax.numpy as jnp
from jax import lax
from jax.experimental import pallas as pl
from jax.experimental.pallas import tpu as pltpu
from jax.experimental.pallas import tpu_sc as plsc

_NC = 2
_NS = 16
_NW = _NC * _NS
_L = 16

_B = 16384
_D = 128
_BPW = _B // _NW
_OFFS = (0, 32, 288, 320, 384)
_VTOT = 416


def _body(idx_hbm, tab_hbm, out_hbm, tab_v, idx_v, out_v):
    wid = lax.axis_index("s") * _NC + lax.axis_index("c")
    base = wid * _BPW

    pltpu.sync_copy(tab_hbm, tab_v)
    for t in range(5):
        pltpu.sync_copy(
            idx_hbm.at[pl.ds(t * _B + base, _BPW)],
            idx_v.at[pl.ds(t * _BPW, _BPW)],
        )

    @plsc.parallel_loop(0, _BPW // _L, unroll=2)
    def _(g):
        ivecs = [idx_v[pl.ds(t * _BPW + g * _L, _L)] for t in range(5)]
        for j in range(_L):
            ob = (g * _L + j) * _D
            bases = [(ivecs[t][j] + _OFFS[t]) * (_D // 2) for t in range(5)]
            vs = [
                plsc.bitcast(
                    tab_v[pl.ds(bases[t] + u * _L, _L)], jnp.bfloat16
                )
                for u in range(_D // (2 * _L))
                for t in range(5)
            ]
            for u in range(_D // (2 * _L)):
                c = vs[5 * u : 5 * u + 5]
                acc = ((c[0] + c[1]) + (c[2] + c[3])) + c[4]
                a, b = plsc.unpack(acc, format=plsc.PackFormat.INTERLEAVED)
                out_v[pl.ds(ob + u * 2 * _L, _L)] = a
                out_v[pl.ds(ob + u * 2 * _L + _L, _L)] = b
    pltpu.sync_copy(out_v, out_hbm.at[pl.ds(base * _D, _BPW * _D)])


@jax.jit
def kernel(x_features_indices, W_exchange, W_pair, W_type, W_feature, W_level):
    idx = x_features_indices.astype(jnp.int32).T.reshape(-1)
    tab = (
        jnp.concatenate([W_exchange, W_pair, W_type, W_feature, W_level], axis=0)
        .astype(jnp.bfloat16)
        .reshape(_VTOT, _D // (2 * _L), 2, _L)
        .transpose(0, 1, 3, 2)
        .reshape(-1, 2)
    )
    tab = lax.bitcast_convert_type(tab, jnp.int32)
    mesh = plsc.VectorSubcoreMesh(
        core_axis_name="c", subcore_axis_name="s",
        num_cores=_NC, num_subcores=_NS,
    )
    f = pl.kernel(
        _body,
        out_type=jax.ShapeDtypeStruct((_B * _D,), jnp.float32),
        mesh=mesh,
        compiler_params=pltpu.CompilerParams(needs_layout_passes=False),
        scratch_types=[
            pltpu.VMEM((_VTOT * _D // 2,), jnp.int32),
            pltpu.VMEM((5 * _BPW,), jnp.int32),
            pltpu.VMEM((_BPW * _D,), jnp.float32),
        ],
    )
    return f(idx, tab).reshape(_B, _D)


# speedup vs baseline: 1.1666x; 1.0474x over previous
import jax
import jax.numpy as jnp
from jax import lax
from jax.experimental import pallas as pl
from jax.experimental.pallas import tpu as pltpu
from jax.experimental.pallas import tpu_sc as plsc

_NC = 2
_NS = 16
_NW = _NC * _NS
_L = 16

_B = 16384
_D = 128
_BPW = _B // _NW
_OFFS = (0, 32, 288, 320, 384)
_VTOT = 416


def _body(idx_hbm, tab_hbm, out_hbm, tab_v, idx_v, out_v, sem_in, sem_out):
    wid = lax.axis_index("s") * _NC + lax.axis_index("c")
    base = wid * _BPW

    cps = [pltpu.async_copy(tab_hbm, tab_v, sem_in)]
    for t in range(5):
        cps.append(
            pltpu.async_copy(
                idx_hbm.at[pl.ds(t * _B + base, _BPW)],
                idx_v.at[pl.ds(t * _BPW, _BPW)],
                sem_in,
            )
        )
    for cp in cps:
        cp.wait()

    def run_groups(g_lo, g_hi):
        @plsc.parallel_loop(g_lo, g_hi, unroll=2)
        def _(g):
            ivecs = [idx_v[pl.ds(t * _BPW + g * _L, _L)] for t in range(5)]
            for j in range(_L):
                ob = (g * _L + j) * _D
                bases = [(ivecs[t][j] + _OFFS[t]) * (_D // 2) for t in range(5)]
                vs = [
                    plsc.bitcast(
                        tab_v[pl.ds(bases[t] + u * _L, _L)], jnp.bfloat16
                    )
                    for u in range(_D // (2 * _L))
                    for t in range(5)
                ]
                for u in range(_D // (2 * _L)):
                    c = vs[5 * u : 5 * u + 5]
                    acc = ((c[0] + c[1]) + (c[2] + c[3])) + c[4]
                    a, b = plsc.unpack(acc, format=plsc.PackFormat.INTERLEAVED)
                    out_v[pl.ds(ob + u * 2 * _L, _L)] = a
                    out_v[pl.ds(ob + u * 2 * _L + _L, _L)] = b

    _HG = _BPW // (2 * _L)
    _HE = _HG * _L * _D
    run_groups(0, _HG)
    cp0 = pltpu.async_copy(
        out_v.at[pl.ds(0, _HE)],
        out_hbm.at[pl.ds(base * _D, _HE)],
        sem_out,
    )
    run_groups(_HG, 2 * _HG)
    cp1 = pltpu.async_copy(
        out_v.at[pl.ds(_HE, _HE)],
        out_hbm.at[pl.ds(base * _D + _HE, _HE)],
        sem_out,
    )
    cp0.wait()
    cp1.wait()


@jax.jit
def kernel(x_features_indices, W_exchange, W_pair, W_type, W_feature, W_level):
    idx = x_features_indices.astype(jnp.int32).T.reshape(-1)
    tab = (
        jnp.concatenate([W_exchange, W_pair, W_type, W_feature, W_level], axis=0)
        .astype(jnp.bfloat16)
        .reshape(_VTOT, _D // (2 * _L), 2, _L)
        .transpose(0, 1, 3, 2)
        .reshape(-1, 2)
    )
    tab = lax.bitcast_convert_type(tab, jnp.int32)
    mesh = plsc.VectorSubcoreMesh(
        core_axis_name="c", subcore_axis_name="s",
        num_cores=_NC, num_subcores=_NS,
    )
    f = pl.kernel(
        _body,
        out_type=jax.ShapeDtypeStruct((_B * _D,), jnp.float32),
        mesh=mesh,
        compiler_params=pltpu.CompilerParams(needs_layout_passes=False),
        scratch_types=[
            pltpu.VMEM((_VTOT * _D // 2,), jnp.int32),
            pltpu.VMEM((5 * _BPW,), jnp.int32),
            pltpu.VMEM((_BPW * _D,), jnp.float32),
            pltpu.SemaphoreType.DMA,
            pltpu.SemaphoreType.DMA,
        ],
    )
    return f(idx, tab).reshape(_B, _D)
